# Initial kernel scaffold; baseline (speedup 1.0000x reference)
import jax
import jax.numpy as jnp
from jax.experimental import pallas as pl


def kernel(inputs, centroids):
    raise NotImplementedError("write your pallas kernel here")


# speedup vs baseline: 2.3318x; 2.3318x over previous
import functools

import jax
import jax.numpy as jnp
from jax import lax
from jax.experimental import pallas as pl
from jax.experimental.pallas import tpu as pltpu
from jax.experimental.pallas import tpu_sc as plsc

NCODEBOOKS = 16
NCENTROIDS = 1024
SUBVECT = 256
BATCH = 4096

TB = 256


def _tc_body(x_ref, c_ref, idx_ref):
    cols = []
    for c in range(NCODEBOOKS):
        a = x_ref[:, c, :]
        w = c_ref[c, :, :]
        resp = lax.dot_general(
            a, w, (((1,), (1,)), ((), ())),
            preferred_element_type=jnp.float32)
        m = jnp.max(resp, axis=1, keepdims=True)
        iota = lax.broadcasted_iota(jnp.int32, resp.shape, 1)
        idx = jnp.min(jnp.where(resp == m, iota, NCENTROIDS), axis=1,
                      keepdims=True)
        cols.append(idx + c * NCENTROIDS)
    idx_ref[...] = jnp.concatenate(cols, axis=1)


def _tc_codes(inputs, centroids):
    grid = (BATCH // TB,)
    return pl.pallas_call(
        _tc_body,
        grid=grid,
        in_specs=[
            pl.BlockSpec((TB, NCODEBOOKS, SUBVECT), lambda i: (i, 0, 0)),
            pl.BlockSpec((NCODEBOOKS, NCENTROIDS, SUBVECT), lambda i: (0, 0, 0)),
        ],
        out_specs=pl.BlockSpec((TB, NCODEBOOKS), lambda i: (i, 0)),
        out_shape=jax.ShapeDtypeStruct((BATCH, NCODEBOOKS), jnp.int32),
    )(inputs, centroids)


def _sc_gather(flat_centroids, idx_flat):
    info = plsc.get_sparse_core_info()
    nw = info.num_cores * info.num_subcores
    total = BATCH * NCODEBOOKS
    b_per_w = total // nw
    chunk = 256
    nchunks = b_per_w // chunk

    mesh = plsc.VectorSubcoreMesh(core_axis_name="c", subcore_axis_name="s")

    @functools.partial(
        pl.kernel,
        out_type=jax.ShapeDtypeStruct((total, SUBVECT), jnp.float32),
        mesh=mesh,
        scratch_types=[
            pltpu.VMEM((b_per_w,), jnp.int32),
            pltpu.VMEM((chunk, SUBVECT), jnp.float32),
            pltpu.SemaphoreType.DMA,
        ],
    )
    def gather(table_hbm, idx_hbm, out_hbm, idx_v, rows_v, sem):
        wid = lax.axis_index("s") * info.num_cores + lax.axis_index("c")
        base = wid * b_per_w
        pltpu.sync_copy(idx_hbm.at[pl.ds(base, b_per_w)], idx_v)
        for j in range(nchunks):
            pltpu.async_copy(
                table_hbm.at[idx_v.at[pl.ds(j * chunk, chunk)]], rows_v,
                sem).wait()
            pltpu.sync_copy(rows_v, out_hbm.at[pl.ds(base + j * chunk, chunk)])

    return gather(flat_centroids, idx_flat)


def kernel(inputs, centroids):
    neighbour_idxs = _tc_codes(inputs, centroids)
    flat_centroids = centroids.reshape((-1, SUBVECT))
    out = _sc_gather(flat_centroids, neighbour_idxs.reshape(-1))
    return (neighbour_idxs, out.reshape((BATCH, NCODEBOOKS, SUBVECT)))


# speedup vs baseline: 2.7028x; 1.1591x over previous
import functools

import jax
import jax.numpy as jnp
from jax import lax
from jax.experimental import pallas as pl
from jax.experimental.pallas import tpu as pltpu
from jax.experimental.pallas import tpu_sc as plsc

NCODEBOOKS = 16
NCENTROIDS = 1024
SUBVECT = 256
BATCH = 4096

TB = 256


def _tc_body(x_ref, c_ref, idx_ref):
    cols = []
    for c in range(NCODEBOOKS):
        a = x_ref[:, c, :]
        w = c_ref[c, :, :]
        resp = lax.dot_general(
            a, w, (((1,), (1,)), ((), ())),
            preferred_element_type=jnp.float32)
        idx = jnp.argmax(resp, axis=1)[:, None].astype(jnp.int32)
        cols.append(idx + c * NCENTROIDS)
    idx_ref[...] = jnp.concatenate(cols, axis=1)


def _tc_codes(inputs, centroids):
    grid = (BATCH // TB,)
    return pl.pallas_call(
        _tc_body,
        grid=grid,
        in_specs=[
            pl.BlockSpec((TB, NCODEBOOKS, SUBVECT), lambda i: (i, 0, 0)),
            pl.BlockSpec((NCODEBOOKS, NCENTROIDS, SUBVECT), lambda i: (0, 0, 0)),
        ],
        out_specs=pl.BlockSpec((TB, NCODEBOOKS), lambda i: (i, 0)),
        out_shape=jax.ShapeDtypeStruct((BATCH, NCODEBOOKS), jnp.int32),
    )(inputs, centroids)


def _sc_gather(flat_centroids, idx_flat):
    info = plsc.get_sparse_core_info()
    nw = info.num_cores * info.num_subcores
    total = BATCH * NCODEBOOKS
    b_per_w = total // nw
    chunk = 256
    nchunks = b_per_w // chunk

    mesh = plsc.VectorSubcoreMesh(core_axis_name="c", subcore_axis_name="s")

    @functools.partial(
        pl.kernel,
        out_type=jax.ShapeDtypeStruct((total, SUBVECT), jnp.float32),
        mesh=mesh,
        scratch_types=[
            pltpu.VMEM((b_per_w,), jnp.int32),
            pltpu.VMEM((chunk, SUBVECT), jnp.float32),
            pltpu.SemaphoreType.DMA,
        ],
    )
    def gather(table_hbm, idx_hbm, out_hbm, idx_v, rows_v, sem):
        wid = lax.axis_index("s") * info.num_cores + lax.axis_index("c")
        base = wid * b_per_w
        pltpu.sync_copy(idx_hbm.at[pl.ds(base, b_per_w)], idx_v)
        for j in range(nchunks):
            pltpu.async_copy(
                table_hbm.at[idx_v.at[pl.ds(j * chunk, chunk)]], rows_v,
                sem).wait()
            pltpu.sync_copy(rows_v, out_hbm.at[pl.ds(base + j * chunk, chunk)])

    return gather(flat_centroids, idx_flat)


def kernel(inputs, centroids):
    neighbour_idxs = _tc_codes(inputs, centroids)
    flat_centroids = centroids.reshape((-1, SUBVECT))
    out = _sc_gather(flat_centroids, neighbour_idxs.reshape(-1))
    return (neighbour_idxs, out.reshape((BATCH, NCODEBOOKS, SUBVECT)))
